# baseline (speedup 1.0000x reference)
import functools
import math

import jax
import jax.numpy as jnp
import numpy as np
from jax import lax
from jax.experimental import pallas as pl
from jax.experimental.pallas import tpu as pltpu
from jax.experimental.pallas import tpu_sc as plsc

N = 10000
E = 320000
D = 128
C = 32
NT = 16
NB = 8
NG = 64
RMAX = 6.0

EP = 327680
CH = 128
EPW = EP // 32
NROWS = 10112
NPAD = 10240
DUMMY = N

TE = 2560
TN = 2048

_MESH = plsc.VectorSubcoreMesh(core_axis_name="c", subcore_axis_name="s")


def _silu(x):
    return x * (1.0 / (1.0 + jnp.exp(-x)))


def _dot(a, b):
    return jnp.dot(a, b, preferred_element_type=jnp.float32)


def _dot_h(a, b):
    return jnp.dot(a, b, preferred_element_type=jnp.float32,
                   precision=lax.Precision.HIGHEST)


def _tile4(x):
    return jnp.concatenate([x, x, x, x], axis=1)


def _sc_gather_feats(fpad_hbm, src2_hbm, dst2_hbm, fs_out, fd_out,
                     idx_s, idx_d, rows_s, rows_d, sem_s, sem_d):
    cid = lax.axis_index("c")
    sid = lax.axis_index("s")
    wid = cid * 16 + sid
    rbase = wid * (EPW // CH)

    def body(j, carry):
        pltpu.sync_copy(src2_hbm.at[rbase + j], idx_s)
        pltpu.sync_copy(dst2_hbm.at[rbase + j], idx_d)
        cs = pltpu.async_copy(fpad_hbm.at[idx_s], rows_s, sem_s)
        cd = pltpu.async_copy(fpad_hbm.at[idx_d], rows_d, sem_d)
        cs.wait()
        cd.wait()
        ebase = (rbase + j) * CH
        pltpu.sync_copy(rows_s, fs_out.at[pl.ds(ebase, CH), :])
        pltpu.sync_copy(rows_d, fd_out.at[pl.ds(ebase, CH), :])
        return carry

    lax.fori_loop(0, EPW // CH, body, 0)


def _run_sc_gather(fpad, src2, dst2):
    return pl.kernel(
        _sc_gather_feats,
        out_type=[jax.ShapeDtypeStruct((EP, 16), jnp.float32),
                  jax.ShapeDtypeStruct((EP, 16), jnp.float32)],
        mesh=_MESH,
        compiler_params=pltpu.CompilerParams(use_tc_tiling_on_sc=False),
        scratch_types=[pltpu.VMEM((CH,), jnp.int32),
                       pltpu.VMEM((CH,), jnp.int32),
                       pltpu.VMEM((CH, 16), jnp.float32),
                       pltpu.VMEM((CH, 16), jnp.float32),
                       pltpu.SemaphoreType.DMA,
                       pltpu.SemaphoreType.DMA],
    )(fpad, src2, dst2)


def _sc_env(msg_hbm, src2_hbm, zeros_hbm, env_out,
            idx0, idx1, rows0, rows1, zbuf, shared,
            smi0, smi1, smm0, smm1, sem):
    cid = lax.axis_index("c")
    sid = lax.axis_index("s")
    wid = cid * 16 + sid

    pltpu.sync_copy(zeros_hbm, zbuf)
    for k in range(-(-(NROWS // CH) // 16)):
        zc = sid + k * 16

        @pl.when(zc < NROWS // CH)
        def _():
            pltpu.sync_copy(zbuf, shared.at[pl.ds(zc * CH, CH), :])
    plsc.subcore_barrier()

    nch = (EP // CH) // 16
    rbase = sid * nch
    idxb = (idx0, idx1)
    rowb = (rows0, rows1)
    semi = (smi0, smi1)
    semm = (smm0, smm1)

    pltpu.async_copy(src2_hbm.at[rbase], idx0, smi0)
    pltpu.async_copy(msg_hbm.at[pl.ds(rbase * CH, CH), :], rows0, smm0)

    def sbody(j, carry):
        c = lax.rem(j, 2)
        row_n = rbase + jnp.minimum(j + 1, nch - 1)

        def pref(nx):
            pltpu.async_copy(src2_hbm.at[row_n], idxb[nx], semi[nx])
            pltpu.async_copy(msg_hbm.at[pl.ds(row_n * CH, CH), :], rowb[nx], semm[nx])

        def scat(c_):
            row_c = rbase + j
            pltpu.make_async_copy(src2_hbm.at[row_c], idxb[c_], semi[c_]).wait()
            pltpu.make_async_copy(msg_hbm.at[pl.ds(row_c * CH, CH), :], rowb[c_], semm[c_]).wait()
            pltpu.sync_copy(rowb[c_], shared.at[idxb[c_]], add=True)

        @pl.when(c == 0)
        def _():
            pref(1)
            scat(0)

        @pl.when(c == 1)
        def _():
            pref(0)
            scat(1)

        return carry

    lax.fori_loop(0, nch, sbody, 0)
    pltpu.make_async_copy(src2_hbm.at[rbase + nch - 1], idx0, smi0).wait()
    pltpu.make_async_copy(msg_hbm.at[pl.ds((rbase + nch - 1) * CH, CH), :], rows0, smm0).wait()
    plsc.subcore_barrier()

    gbase = wid * (EPW // CH)

    def gbody(j, carry):
        pltpu.sync_copy(src2_hbm.at[gbase + j], idx0)
        pltpu.async_copy(shared.at[idx0], rows0, sem).wait()
        pltpu.sync_copy(rows0, env_out.at[pl.ds((gbase + j) * CH, CH), :])
        return carry

    lax.fori_loop(0, EPW // CH, gbody, 0)


def _run_sc_env(msg, src2, zeros128):
    return pl.kernel(
        _sc_env,
        out_type=jax.ShapeDtypeStruct((EP, 128), jnp.float32),
        mesh=_MESH,
        scratch_types=[pltpu.VMEM((CH,), jnp.int32),
                       pltpu.VMEM((CH,), jnp.int32),
                       pltpu.VMEM((CH, 128), jnp.float32),
                       pltpu.VMEM((CH, 128), jnp.float32),
                       pltpu.VMEM((CH, 128), jnp.float32),
                       pltpu.VMEM_SHARED((NROWS, 128), jnp.float32),
                       pltpu.SemaphoreType.DMA,
                       pltpu.SemaphoreType.DMA,
                       pltpu.SemaphoreType.DMA,
                       pltpu.SemaphoreType.DMA,
                       pltpu.SemaphoreType.DMA],
    )(msg, src2, zeros128)


def _stage_a(fs_ref, fd_ref, w0_ref, wtype_ref, b0_ref, w1_ref, w2_ref,
             wspht_ref, wenv1t_ref, k4_ref, xs_out, msg_out, aux_out):
    fs = fs_ref[...]
    fd = fd_ref[...]
    v0 = fd[:, 0:1] - fs[:, 0:1]
    v1 = fd[:, 1:2] - fs[:, 1:2]
    v2 = fd[:, 2:3] - fs[:, 2:3]
    r2 = v0 * v0 + v1 * v1 + v2 * v2
    r = jnp.sqrt(r2 + 1e-8)
    inv_r = 1.0 / r
    u = jnp.clip(r * (1.0 / RMAX), 0.0, 1.0)
    u2 = u * u
    u6 = u2 * u2 * u2
    u7 = u6 * u
    u8 = u7 * u
    fcut = 1.0 - 28.0 * u6 + 48.0 * u7 - 21.0 * u8
    n = (lax.broadcasted_iota(jnp.int32, (1, NB), 1) + 1).astype(jnp.float32)
    bess = jnp.sin(r * (n * (math.pi / RMAX))) * (math.sqrt(2.0 / RMAX) * inv_r)
    radial = bess * fcut
    oh_s = (fs[:, 3:4].astype(jnp.int32) == lax.broadcasted_iota(jnp.int32, (TE, NT), 1)).astype(jnp.float32)
    oh_d = (fd[:, 3:4].astype(jnp.int32) == lax.broadcasted_iota(jnp.int32, (TE, NT), 1)).astype(jnp.float32)
    emb_s = _dot_h(oh_s, wtype_ref[...])
    emb_d = _dot_h(oh_d, wtype_ref[...])
    feat0 = jnp.concatenate([emb_s, emb_d, radial], axis=1)
    h = _dot(feat0, w0_ref[...]) + b0_ref[...]
    x = _silu(h)
    x = _silu(_dot(x, w1_ref[...]))
    x = _silu(_dot(x, w2_ref[...]))
    wcht = _dot(x, wspht_ref[...])
    e1t = _dot(x, wenv1t_ref[...])
    sh4 = jnp.concatenate([jnp.ones((TE, 1), jnp.float32), v0 * inv_r,
                           v1 * inv_r, v2 * inv_r], axis=1)
    shfull = _dot_h(sh4, k4_ref[...])
    v_full = wcht * shfull
    msg = e1t * v_full * (1.0 / 32.0)
    xs_out[...] = x
    msg_out[...] = msg
    aux_out[...] = jnp.concatenate(
        [fcut, v0 * inv_r, v1 * inv_r, v2 * inv_r,
         jnp.zeros((TE, 4), jnp.float32)], axis=1)


def _stage_b(xs_ref, aux_ref, env_ref, wlat_ref, wupdt_ref, wenv2t_ref, s_ref,
             wspht_ref, k4_ref, xs2_out, v2_out, msg2_out):
    x = xs_ref[...]
    aux = aux_ref[...]
    sh4 = jnp.concatenate([jnp.ones((TE, 1), jnp.float32), aux[:, 1:2],
                           aux[:, 2:3], aux[:, 3:4]], axis=1)
    v_full = _dot(x, wspht_ref[...]) * _dot_h(sh4, k4_ref[...])
    env = env_ref[...]
    prod = v_full * env
    inv = _dot_h(prod, s_ref[...])
    x2 = x + _silu(_dot(jnp.concatenate([x, inv], axis=1), wlat_ref[...]))
    updt = _dot(x2, wupdt_ref[...])
    v2 = v_full * updt + env
    ew2t = _dot(x2, wenv2t_ref[...])
    msg2 = ew2t * v2 * (1.0 / 32.0)
    xs2_out[...] = x2
    v2_out[...] = v2
    msg2_out[...] = msg2


def _stage_c(xs2_ref, v2_ref, env2_ref, fd_ref, aux_ref, wlat2_ref,
             wout_ref, scales_ref, s_ref, acc_out):
    i = pl.program_id(0)
    x2 = xs2_ref[...]
    v2 = v2_ref[...]
    env2 = env2_ref[...]
    prod = v2 * env2
    inv2 = _dot_h(prod, s_ref[...])
    x3 = x2 + _silu(_dot(jnp.concatenate([x2, inv2], axis=1), wlat2_ref[...]))
    e = _dot(x3, wout_ref[...])
    fcut = aux_ref[:, 0:1]
    td = fd_ref[:, 3:4]
    gd = fd_ref[:, 4:5]
    oh_t = (td.astype(jnp.int32) == lax.broadcasted_iota(jnp.int32, (TE, NT), 1)).astype(jnp.float32)
    scale_d = _dot_h(oh_t, scales_ref[...])
    ev = e * fcut * scale_d
    oh_g = (gd.astype(jnp.int32) == lax.broadcasted_iota(jnp.int32, (TE, NG), 1)).astype(jnp.float32)
    part = lax.dot_general(oh_g, ev, (((0,), (0,)), ((), ())),
                           preferred_element_type=jnp.float32,
                           precision=lax.Precision.HIGHEST)

    @pl.when(i == 0)
    def _():
        acc_out[...] = jnp.zeros((1, NG), jnp.float32)

    acc_out[...] += part.reshape(1, NG)


def _atom_shift(f_ref, shifts_ref, acc_out):
    i = pl.program_id(0)
    tf = f_ref[:, 3:4]
    gf = f_ref[:, 4:5]
    oh_t = (tf.astype(jnp.int32) == lax.broadcasted_iota(jnp.int32, (TN, NT), 1)).astype(jnp.float32)
    sval = _dot_h(oh_t, shifts_ref[...])
    oh_g = (gf.astype(jnp.int32) == lax.broadcasted_iota(jnp.int32, (TN, NG), 1)).astype(jnp.float32)
    part = lax.dot_general(oh_g, sval, (((0,), (0,)), ((), ())),
                           preferred_element_type=jnp.float32,
                           precision=lax.Precision.HIGHEST)

    @pl.when(i == 0)
    def _():
        acc_out[...] = jnp.zeros((1, NG), jnp.float32)

    acc_out[...] += part.reshape(1, NG)


def _edge_spec():
    return pl.BlockSpec((TE, 128), lambda i: (i, 0))


def _env_spec():
    return pl.BlockSpec((TE, 128), lambda i: (i, 0))


def _full(shape):
    return pl.BlockSpec(shape, lambda i: tuple(0 for _ in shape))


def _run_stage_a(fs, fd, w0, wtype, b0, w1, w2, wspht, wenv1t, k4):
    grid = (EP // TE,)
    return pl.pallas_call(
        _stage_a,
        grid=grid,
        in_specs=[pl.BlockSpec((TE, 16), lambda i: (i, 0)),
                  pl.BlockSpec((TE, 16), lambda i: (i, 0)),
                  _full((2 * 32 + NB, D)), _full((NT, 32)),
                  _full((1, D)), _full((D, D)), _full((D, D)),
                  _full((D, D)), _full((D, D)), _full((4, D))],
        out_specs=[_edge_spec(), _edge_spec(),
                   pl.BlockSpec((TE, 8), lambda i: (i, 0))],
        out_shape=[jax.ShapeDtypeStruct((EP, 128), jnp.float32),
                   jax.ShapeDtypeStruct((EP, 128), jnp.float32),
                   jax.ShapeDtypeStruct((EP, 8), jnp.float32)],
    )(fs, fd, w0, wtype, b0, w1, w2, wspht, wenv1t, k4)


def _run_stage_b(xs, aux, env3, wlat, wupdt, wenv2t, smat, wspht, k4):
    grid = (EP // TE,)
    return pl.pallas_call(
        _stage_b,
        grid=grid,
        in_specs=[_edge_spec(), pl.BlockSpec((TE, 8), lambda i: (i, 0)),
                  _env_spec(),
                  _full((D + C, D)), _full((D, D)), _full((D, D)),
                  _full((D, C)), _full((D, D)), _full((4, D))],
        out_specs=[_edge_spec(), _edge_spec(), _edge_spec()],
        out_shape=[jax.ShapeDtypeStruct((EP, 128), jnp.float32),
                   jax.ShapeDtypeStruct((EP, 128), jnp.float32),
                   jax.ShapeDtypeStruct((EP, 128), jnp.float32)],
    )(xs, aux, env3, wlat, wupdt, wenv2t, smat, wspht, k4)


def _run_stage_c(xs2, v2, env3, fd, aux, wlat2, wout, scales, smat):
    grid = (EP // TE,)
    return pl.pallas_call(
        _stage_c,
        grid=grid,
        in_specs=[_edge_spec(), _edge_spec(), _env_spec(),
                  pl.BlockSpec((TE, 16), lambda i: (i, 0)),
                  pl.BlockSpec((TE, 8), lambda i: (i, 0)),
                  _full((D + C, D)), _full((D, 1)), _full((NT, 1)),
                  _full((D, C))],
        out_specs=pl.BlockSpec((1, NG), lambda i: (0, 0)),
        out_shape=jax.ShapeDtypeStruct((1, NG), jnp.float32),
    )(xs2, v2, env3, fd, aux, wlat2, wout, scales, smat)


def _run_atom_shift(f_pad, shifts):
    grid = (NPAD // TN,)
    return pl.pallas_call(
        _atom_shift,
        grid=grid,
        in_specs=[pl.BlockSpec((TN, 16), lambda i: (i, 0)), _full((NT, 1))],
        out_specs=pl.BlockSpec((1, NG), lambda i: (0, 0)),
        out_shape=jax.ShapeDtypeStruct((1, NG), jnp.float32),
    )(f_pad, shifts)


def kernel(pos, W_type, W0, b0, W1, W2, W_sph, W_env1, W_lat1, W_upd1,
           W_env2, W_lat2, W_upd2, W_out, scales, shifts,
           edge_index, atom_types, batch):
    src = edge_index[0].astype(jnp.int32)
    dst = edge_index[1].astype(jnp.int32)

    tf = atom_types.astype(jnp.float32)[:, None]
    gf = batch.astype(jnp.float32)[:, None]
    f_tab = jnp.concatenate([pos, tf, gf, jnp.zeros((N, 11), jnp.float32)], axis=1)
    pad_rows = jnp.zeros((NPAD - N, 16), jnp.float32).at[:, 4].set(float(NG))
    f_pad = jnp.concatenate([f_tab, pad_rows], axis=0)

    padv = jnp.full((EP - E,), DUMMY, jnp.int32)
    src2 = jnp.concatenate([src, padv]).reshape(EP // CH, CH)
    dst2 = jnp.concatenate([dst, padv]).reshape(EP // CH, CH)
    zeros128 = jnp.zeros((CH, 128), jnp.float32)

    b0r = b0.reshape(1, D)
    wspht = jnp.tile(W_sph, (1, 4))
    wenv1t = jnp.tile(W_env1, (1, 4))
    wupd1t = jnp.tile(W_upd1, (1, 4))
    wenv2t = jnp.tile(W_env2, (1, 4))
    k4np = np.zeros((4, D), np.float32)
    for k in range(4):
        k4np[k, 32 * k:32 * (k + 1)] = 1.0
    k4 = jnp.asarray(k4np)
    snp = np.zeros((D, C), np.float32)
    for k in range(4):
        for c in range(C):
            snp[k * 32 + c, c] = 1.0
    smat = jnp.asarray(snp)

    fs, fd = _run_sc_gather(f_pad, src2, dst2)

    xs, msg1, aux = _run_stage_a(fs, fd, W0, W_type, b0r,
                                 W1, W2, wspht, wenv1t, k4)

    env3_1 = _run_sc_env(msg1, src2, zeros128)

    xs2, v2, msg2 = _run_stage_b(xs, aux, env3_1, W_lat1, wupd1t, wenv2t, smat, wspht, k4)

    env3_2 = _run_sc_env(msg2, src2, zeros128)

    acc_e = _run_stage_c(xs2, v2, env3_2, fd, aux, W_lat2, W_out,
                         scales.reshape(NT, 1), smat)
    acc_a = _run_atom_shift(f_pad, shifts.reshape(NT, 1))
    return (acc_e + acc_a).reshape(NG)


# speedup vs baseline: 1.0757x; 1.0757x over previous
import math

import jax
import jax.numpy as jnp
from jax import lax
from jax.experimental import pallas as pl
from jax.experimental.pallas import tpu as pltpu
from jax.experimental.pallas import tpu_sc as plsc

N = 10000
E = 320000
D = 128
C = 32
NT = 16
NB = 8
NG = 64
RMAX = 6.0

EP = 327680
CH = 128
EPW = EP // 32
NROWS = 10112
NPAD = 10240
DUMMY = N

TE = 1280
TN = 2048

_MESH = plsc.VectorSubcoreMesh(core_axis_name="c", subcore_axis_name="s")


def _silu(x):
    return x * (1.0 / (1.0 + jnp.exp(-x)))


def _dot(a, b):
    return jnp.dot(a, b, preferred_element_type=jnp.float32)


def _dot_h(a, b):
    return jnp.dot(a, b, preferred_element_type=jnp.float32,
                   precision=lax.Precision.HIGHEST)


def _tile4(x):
    return jnp.concatenate([x, x, x, x], axis=1)


def _sc_gather_feats(fpad_hbm, src2_hbm, dst2_hbm, fs_out, fd_out,
                     idx_s, idx_d, rows_s, rows_d, sem_s, sem_d):
    cid = lax.axis_index("c")
    sid = lax.axis_index("s")
    wid = cid * 16 + sid
    rbase = wid * (EPW // CH)

    def body(j, carry):
        pltpu.sync_copy(src2_hbm.at[rbase + j], idx_s)
        pltpu.sync_copy(dst2_hbm.at[rbase + j], idx_d)
        cs = pltpu.async_copy(fpad_hbm.at[idx_s], rows_s, sem_s)
        cd = pltpu.async_copy(fpad_hbm.at[idx_d], rows_d, sem_d)
        cs.wait()
        cd.wait()
        ebase = (rbase + j) * CH
        pltpu.sync_copy(rows_s, fs_out.at[pl.ds(ebase, CH), :])
        pltpu.sync_copy(rows_d, fd_out.at[pl.ds(ebase, CH), :])
        return carry

    lax.fori_loop(0, EPW // CH, body, 0)


def _run_sc_gather(fpad, src2, dst2):
    return pl.kernel(
        _sc_gather_feats,
        out_type=[jax.ShapeDtypeStruct((EP, 16), jnp.float32),
                  jax.ShapeDtypeStruct((EP, 16), jnp.float32)],
        mesh=_MESH,
        compiler_params=pltpu.CompilerParams(use_tc_tiling_on_sc=False),
        scratch_types=[pltpu.VMEM((CH,), jnp.int32),
                       pltpu.VMEM((CH,), jnp.int32),
                       pltpu.VMEM((CH, 16), jnp.float32),
                       pltpu.VMEM((CH, 16), jnp.float32),
                       pltpu.SemaphoreType.DMA,
                       pltpu.SemaphoreType.DMA],
    )(fpad, src2, dst2)


def _sc_env(msg_hbm, src2_hbm, zeros_hbm, env_out,
            idx0, idx1, rows0, rows1, zbuf, shared,
            smi0, smi1, smm0, smm1, sem):
    cid = lax.axis_index("c")
    sid = lax.axis_index("s")
    wid = cid * 16 + sid

    pltpu.sync_copy(zeros_hbm, zbuf)
    for k in range(-(-(NROWS // CH) // 16)):
        zc = sid + k * 16

        @pl.when(zc < NROWS // CH)
        def _():
            pltpu.sync_copy(zbuf, shared.at[pl.ds(zc * CH, CH), :])
    plsc.subcore_barrier()

    nch = (EP // CH) // 16
    rbase = sid * nch
    idxb = (idx0, idx1)
    rowb = (rows0, rows1)
    semi = (smi0, smi1)
    semm = (smm0, smm1)

    pltpu.async_copy(src2_hbm.at[rbase], idx0, smi0)
    pltpu.async_copy(msg_hbm.at[pl.ds(rbase * CH, CH), :], rows0, smm0)

    def sbody(j, carry):
        c = lax.rem(j, 2)
        row_n = rbase + jnp.minimum(j + 1, nch - 1)

        def pref(nx):
            pltpu.async_copy(src2_hbm.at[row_n], idxb[nx], semi[nx])
            pltpu.async_copy(msg_hbm.at[pl.ds(row_n * CH, CH), :], rowb[nx], semm[nx])

        def scat(c_):
            row_c = rbase + j
            pltpu.make_async_copy(src2_hbm.at[row_c], idxb[c_], semi[c_]).wait()
            pltpu.make_async_copy(msg_hbm.at[pl.ds(row_c * CH, CH), :], rowb[c_], semm[c_]).wait()
            pltpu.sync_copy(rowb[c_], shared.at[idxb[c_]], add=True)

        @pl.when(c == 0)
        def _():
            pref(1)
            scat(0)

        @pl.when(c == 1)
        def _():
            pref(0)
            scat(1)

        return carry

    lax.fori_loop(0, nch, sbody, 0)
    pltpu.make_async_copy(src2_hbm.at[rbase + nch - 1], idx0, smi0).wait()
    pltpu.make_async_copy(msg_hbm.at[pl.ds((rbase + nch - 1) * CH, CH), :], rows0, smm0).wait()
    plsc.subcore_barrier()

    gbase = wid * (EPW // CH)

    def gbody(j, carry):
        pltpu.sync_copy(src2_hbm.at[gbase + j], idx0)
        pltpu.async_copy(shared.at[idx0], rows0, sem).wait()
        pltpu.sync_copy(rows0, env_out.at[pl.ds((gbase + j) * CH, CH), :])
        return carry

    lax.fori_loop(0, EPW // CH, gbody, 0)


def _run_sc_env(msg, src2, zeros128):
    return pl.kernel(
        _sc_env,
        out_type=jax.ShapeDtypeStruct((EP, 128), jnp.float32),
        mesh=_MESH,
        scratch_types=[pltpu.VMEM((CH,), jnp.int32),
                       pltpu.VMEM((CH,), jnp.int32),
                       pltpu.VMEM((CH, 128), jnp.float32),
                       pltpu.VMEM((CH, 128), jnp.float32),
                       pltpu.VMEM((CH, 128), jnp.float32),
                       pltpu.VMEM_SHARED((NROWS, 128), jnp.float32),
                       pltpu.SemaphoreType.DMA,
                       pltpu.SemaphoreType.DMA,
                       pltpu.SemaphoreType.DMA,
                       pltpu.SemaphoreType.DMA,
                       pltpu.SemaphoreType.DMA],
    )(msg, src2, zeros128)


def _stage_a(fs_ref, fd_ref, w0_ref, wtype_ref, b0_ref, w1_ref, w2_ref,
             wsph_ref, wenv1_ref, xs_out, v_out, msg_out, aux_out):
    fs = fs_ref[...]
    fd = fd_ref[...]
    v0 = fd[:, 0:1] - fs[:, 0:1]
    v1 = fd[:, 1:2] - fs[:, 1:2]
    v2 = fd[:, 2:3] - fs[:, 2:3]
    r2 = v0 * v0 + v1 * v1 + v2 * v2
    r = jnp.sqrt(r2 + 1e-8)
    inv_r = 1.0 / r
    u = jnp.clip(r * (1.0 / RMAX), 0.0, 1.0)
    u2 = u * u
    u6 = u2 * u2 * u2
    u7 = u6 * u
    u8 = u7 * u
    fcut = 1.0 - 28.0 * u6 + 48.0 * u7 - 21.0 * u8
    n = (lax.broadcasted_iota(jnp.int32, (1, NB), 1) + 1).astype(jnp.float32)
    bess = jnp.sin(r * (n * (math.pi / RMAX))) * (math.sqrt(2.0 / RMAX) * inv_r)
    radial = bess * fcut
    oh_s = (fs[:, 3:4].astype(jnp.int32) == lax.broadcasted_iota(jnp.int32, (TE, NT), 1)).astype(jnp.float32)
    oh_d = (fd[:, 3:4].astype(jnp.int32) == lax.broadcasted_iota(jnp.int32, (TE, NT), 1)).astype(jnp.float32)
    emb_s = _dot_h(oh_s, wtype_ref[...])
    emb_d = _dot_h(oh_d, wtype_ref[...])
    feat0 = jnp.concatenate([emb_s, emb_d, radial], axis=1)
    h = _dot(feat0, w0_ref[...]) + b0_ref[...]
    x = _silu(h)
    x = _silu(_dot(x, w1_ref[...]))
    x = _silu(_dot(x, w2_ref[...]))
    wch = _dot(x, wsph_ref[...])
    e1 = _dot(x, wenv1_ref[...])
    v_full = jnp.concatenate([wch, wch * (v0 * inv_r), wch * (v1 * inv_r),
                              wch * (v2 * inv_r)], axis=1)
    msg = _tile4(e1) * v_full * (1.0 / 32.0)
    xs_out[...] = x
    v_out[...] = v_full
    msg_out[...] = msg
    aux_out[...] = fcut + jnp.zeros((TE, 8), jnp.float32)


def _stage_b(xs_ref, v_ref, env_ref, wlat_ref, wupd_ref, wenv2_ref,
             xs2_out, v2_out, msg2_out):
    x = xs_ref[...]
    v_full = v_ref[...]
    env = env_ref[...]
    prod = v_full * env
    inv = prod[:, 0:32] + prod[:, 32:64] + prod[:, 64:96] + prod[:, 96:128]
    x2 = x + _silu(_dot(jnp.concatenate([x, inv], axis=1), wlat_ref[...]))
    upd = _dot(x2, wupd_ref[...])
    v2 = v_full * _tile4(upd) + env
    ew2 = _dot(x2, wenv2_ref[...])
    msg2 = _tile4(ew2) * v2 * (1.0 / 32.0)
    xs2_out[...] = x2
    v2_out[...] = v2
    msg2_out[...] = msg2


def _stage_c(xs2_ref, v2_ref, env2_ref, fd_ref, aux_ref, wlat2_ref,
             wout_ref, scales_ref, acc_out):
    i = pl.program_id(0)
    x2 = xs2_ref[...]
    v2 = v2_ref[...]
    env2 = env2_ref[...]
    prod = v2 * env2
    inv2 = prod[:, 0:32] + prod[:, 32:64] + prod[:, 64:96] + prod[:, 96:128]
    x3 = x2 + _silu(_dot(jnp.concatenate([x2, inv2], axis=1), wlat2_ref[...]))
    e = _dot(x3, wout_ref[...])
    fcut = aux_ref[:, 0:1]
    td = fd_ref[:, 3:4]
    gd = fd_ref[:, 4:5]
    oh_t = (td.astype(jnp.int32) == lax.broadcasted_iota(jnp.int32, (TE, NT), 1)).astype(jnp.float32)
    scale_d = _dot_h(oh_t, scales_ref[...])
    ev = e * fcut * scale_d
    oh_g = (gd.astype(jnp.int32) == lax.broadcasted_iota(jnp.int32, (TE, NG), 1)).astype(jnp.float32)
    part = lax.dot_general(oh_g, ev, (((0,), (0,)), ((), ())),
                           preferred_element_type=jnp.float32,
                           precision=lax.Precision.HIGHEST)

    @pl.when(i == 0)
    def _():
        acc_out[...] = jnp.zeros((1, NG), jnp.float32)

    acc_out[...] += part.reshape(1, NG)


def _atom_shift(f_ref, shifts_ref, acc_out):
    i = pl.program_id(0)
    tf = f_ref[:, 3:4]
    gf = f_ref[:, 4:5]
    oh_t = (tf.astype(jnp.int32) == lax.broadcasted_iota(jnp.int32, (TN, NT), 1)).astype(jnp.float32)
    sval = _dot_h(oh_t, shifts_ref[...])
    oh_g = (gf.astype(jnp.int32) == lax.broadcasted_iota(jnp.int32, (TN, NG), 1)).astype(jnp.float32)
    part = lax.dot_general(oh_g, sval, (((0,), (0,)), ((), ())),
                           preferred_element_type=jnp.float32,
                           precision=lax.Precision.HIGHEST)

    @pl.when(i == 0)
    def _():
        acc_out[...] = jnp.zeros((1, NG), jnp.float32)

    acc_out[...] += part.reshape(1, NG)


def _edge_spec():
    return pl.BlockSpec((TE, 128), lambda i: (i, 0))


def _full(shape):
    return pl.BlockSpec(shape, lambda i: tuple(0 for _ in shape))


def _run_stage_a(fs, fd, w0, wtype, b0, w1, w2, wsph, wenv1):
    grid = (EP // TE,)
    return pl.pallas_call(
        _stage_a,
        grid=grid,
        in_specs=[pl.BlockSpec((TE, 16), lambda i: (i, 0)),
                  pl.BlockSpec((TE, 16), lambda i: (i, 0)),
                  _full((2 * 32 + NB, D)), _full((NT, 32)),
                  _full((1, D)), _full((D, D)), _full((D, D)),
                  _full((D, C)), _full((D, C))],
        out_specs=[_edge_spec(), _edge_spec(), _edge_spec(),
                   pl.BlockSpec((TE, 8), lambda i: (i, 0))],
        out_shape=[jax.ShapeDtypeStruct((EP, 128), jnp.float32),
                   jax.ShapeDtypeStruct((EP, 128), jnp.float32),
                   jax.ShapeDtypeStruct((EP, 128), jnp.float32),
                   jax.ShapeDtypeStruct((EP, 8), jnp.float32)],
    )(fs, fd, w0, wtype, b0, w1, w2, wsph, wenv1)


def _run_stage_b(xs, v, env3, wlat, wupd, wenv2):
    grid = (EP // TE,)
    return pl.pallas_call(
        _stage_b,
        grid=grid,
        in_specs=[_edge_spec(), _edge_spec(), _edge_spec(),
                  _full((D + C, D)), _full((D, C)), _full((D, C))],
        out_specs=[_edge_spec(), _edge_spec(), _edge_spec()],
        out_shape=[jax.ShapeDtypeStruct((EP, 128), jnp.float32),
                   jax.ShapeDtypeStruct((EP, 128), jnp.float32),
                   jax.ShapeDtypeStruct((EP, 128), jnp.float32)],
    )(xs, v, env3, wlat, wupd, wenv2)


def _run_stage_c(xs2, v2, env3, fd, aux, wlat2, wout, scales):
    grid = (EP // TE,)
    return pl.pallas_call(
        _stage_c,
        grid=grid,
        in_specs=[_edge_spec(), _edge_spec(), _edge_spec(),
                  pl.BlockSpec((TE, 16), lambda i: (i, 0)),
                  pl.BlockSpec((TE, 8), lambda i: (i, 0)),
                  _full((D + C, D)), _full((D, 1)), _full((NT, 1))],
        out_specs=pl.BlockSpec((1, NG), lambda i: (0, 0)),
        out_shape=jax.ShapeDtypeStruct((1, NG), jnp.float32),
    )(xs2, v2, env3, fd, aux, wlat2, wout, scales)


def _run_atom_shift(f_pad, shifts):
    grid = (NPAD // TN,)
    return pl.pallas_call(
        _atom_shift,
        grid=grid,
        in_specs=[pl.BlockSpec((TN, 16), lambda i: (i, 0)), _full((NT, 1))],
        out_specs=pl.BlockSpec((1, NG), lambda i: (0, 0)),
        out_shape=jax.ShapeDtypeStruct((1, NG), jnp.float32),
    )(f_pad, shifts)


def kernel(pos, W_type, W0, b0, W1, W2, W_sph, W_env1, W_lat1, W_upd1,
           W_env2, W_lat2, W_upd2, W_out, scales, shifts,
           edge_index, atom_types, batch):
    src = edge_index[0].astype(jnp.int32)
    dst = edge_index[1].astype(jnp.int32)

    tf = atom_types.astype(jnp.float32)[:, None]
    gf = batch.astype(jnp.float32)[:, None]
    f_tab = jnp.concatenate([pos, tf, gf, jnp.zeros((N, 11), jnp.float32)], axis=1)
    pad_rows = jnp.zeros((NPAD - N, 16), jnp.float32).at[:, 4].set(float(NG))
    f_pad = jnp.concatenate([f_tab, pad_rows], axis=0)

    padv = jnp.full((EP - E,), DUMMY, jnp.int32)
    src2 = jnp.concatenate([src, padv]).reshape(EP // CH, CH)
    dst2 = jnp.concatenate([dst, padv]).reshape(EP // CH, CH)
    zeros128 = jnp.zeros((CH, 128), jnp.float32)

    b0r = b0.reshape(1, D)

    fs, fd = _run_sc_gather(f_pad, src2, dst2)

    xs, v, msg1, aux = _run_stage_a(fs, fd, W0, W_type, b0r,
                                    W1, W2, W_sph, W_env1)

    env3_1 = _run_sc_env(msg1, src2, zeros128)

    xs2, v2, msg2 = _run_stage_b(xs, v, env3_1, W_lat1, W_upd1, W_env2)

    env3_2 = _run_sc_env(msg2, src2, zeros128)

    acc_e = _run_stage_c(xs2, v2, env3_2, fd, aux, W_lat2, W_out,
                         scales.reshape(NT, 1))
    acc_a = _run_atom_shift(f_pad, shifts.reshape(NT, 1))
    return (acc_e + acc_a).reshape(NG)


# speedup vs baseline: 1.1266x; 1.0473x over previous
import math

import jax
import jax.numpy as jnp
from jax import lax
from jax.experimental import pallas as pl
from jax.experimental.pallas import tpu as pltpu
from jax.experimental.pallas import tpu_sc as plsc

N = 10000
E = 320000
D = 128
C = 32
NT = 16
NB = 8
NG = 64
RMAX = 6.0

EP = 327680
CH = 128
EPW = EP // 32
NROWS = 10112
NPAD = 10240
DUMMY = N

TE = 1280
TN = 2048

_MESH = plsc.VectorSubcoreMesh(core_axis_name="c", subcore_axis_name="s")


def _silu(x):
    return x * (1.0 / (1.0 + jnp.exp(-x)))


def _dot(a, b):
    return jnp.dot(a, b, preferred_element_type=jnp.float32)


def _dot_h(a, b):
    return jnp.dot(a, b, preferred_element_type=jnp.float32,
                   precision=lax.Precision.HIGHEST)


def _tile4(x):
    return jnp.concatenate([x, x, x, x], axis=1)


def _sc_gather_feats(fpad_hbm, src2_hbm, dst2_hbm, fs_out, fd_out,
                     idx_s, idx_d, rows_s, rows_d, sem_s, sem_d):
    cid = lax.axis_index("c")
    sid = lax.axis_index("s")
    wid = cid * 16 + sid
    rbase = wid * (EPW // CH)

    def body(j, carry):
        pltpu.sync_copy(src2_hbm.at[rbase + j], idx_s)
        pltpu.sync_copy(dst2_hbm.at[rbase + j], idx_d)
        cs = pltpu.async_copy(fpad_hbm.at[idx_s], rows_s, sem_s)
        cd = pltpu.async_copy(fpad_hbm.at[idx_d], rows_d, sem_d)
        cs.wait()
        cd.wait()
        ebase = (rbase + j) * CH
        pltpu.sync_copy(rows_s, fs_out.at[pl.ds(ebase, CH), :])
        pltpu.sync_copy(rows_d, fd_out.at[pl.ds(ebase, CH), :])
        return carry

    lax.fori_loop(0, EPW // CH, body, 0)


def _run_sc_gather(fpad, src2, dst2):
    return pl.kernel(
        _sc_gather_feats,
        out_type=[jax.ShapeDtypeStruct((EP, 16), jnp.float32),
                  jax.ShapeDtypeStruct((EP, 16), jnp.float32)],
        mesh=_MESH,
        compiler_params=pltpu.CompilerParams(use_tc_tiling_on_sc=False),
        scratch_types=[pltpu.VMEM((CH,), jnp.int32),
                       pltpu.VMEM((CH,), jnp.int32),
                       pltpu.VMEM((CH, 16), jnp.float32),
                       pltpu.VMEM((CH, 16), jnp.float32),
                       pltpu.SemaphoreType.DMA,
                       pltpu.SemaphoreType.DMA],
    )(fpad, src2, dst2)


def _sc_env(msg_hbm, src2_hbm, zeros_hbm, env_out,
            idx0, idx1, rows0, rows1, zbuf, shared,
            smi0, smi1, smm0, smm1, sem):
    cid = lax.axis_index("c")
    sid = lax.axis_index("s")
    wid = cid * 16 + sid

    pltpu.sync_copy(zeros_hbm, zbuf)
    for k in range(-(-(NROWS // CH) // 16)):
        zc = sid + k * 16

        @pl.when(zc < NROWS // CH)
        def _():
            pltpu.sync_copy(zbuf, shared.at[pl.ds(zc * CH, CH), :])
    plsc.subcore_barrier()

    nch = (EP // CH) // 16
    rbase = sid * nch
    idxb = (idx0, idx1)
    rowb = (rows0, rows1)
    semi = (smi0, smi1)
    semm = (smm0, smm1)

    pltpu.async_copy(src2_hbm.at[rbase], idx0, smi0)
    pltpu.async_copy(msg_hbm.at[pl.ds(rbase * CH, CH), :], rows0, smm0)

    def sbody(j, carry):
        c = lax.rem(j, 2)
        row_n = rbase + jnp.minimum(j + 1, nch - 1)

        def pref(nx):
            pltpu.async_copy(src2_hbm.at[row_n], idxb[nx], semi[nx])
            pltpu.async_copy(msg_hbm.at[pl.ds(row_n * CH, CH), :], rowb[nx], semm[nx])

        def scat(c_):
            row_c = rbase + j
            pltpu.make_async_copy(src2_hbm.at[row_c], idxb[c_], semi[c_]).wait()
            pltpu.make_async_copy(msg_hbm.at[pl.ds(row_c * CH, CH), :], rowb[c_], semm[c_]).wait()
            pltpu.sync_copy(rowb[c_], shared.at[idxb[c_]], add=True)

        @pl.when(c == 0)
        def _():
            pref(1)
            scat(0)

        @pl.when(c == 1)
        def _():
            pref(0)
            scat(1)

        return carry

    lax.fori_loop(0, nch, sbody, 0)
    pltpu.make_async_copy(src2_hbm.at[rbase + nch - 1], idx0, smi0).wait()
    pltpu.make_async_copy(msg_hbm.at[pl.ds((rbase + nch - 1) * CH, CH), :], rows0, smm0).wait()
    plsc.subcore_barrier()

    ngch = EPW // CH
    gbase = wid * ngch

    pltpu.async_copy(src2_hbm.at[gbase], idx0, smi0)

    def gbody(j, carry):
        c = lax.rem(j, 2)
        row_n = gbase + jnp.minimum(j + 1, ngch - 1)

        def step(c_, nx):
            pltpu.async_copy(src2_hbm.at[row_n], idxb[nx], semi[nx])

            @pl.when(j >= 2)
            def _():
                pltpu.make_async_copy(
                    rowb[c_], env_out.at[pl.ds((gbase + j - 2) * CH, CH), :],
                    semm[c_]).wait()

            pltpu.make_async_copy(src2_hbm.at[gbase + j], idxb[c_], semi[c_]).wait()
            pltpu.async_copy(shared.at[idxb[c_]], rowb[c_], sem).wait()
            pltpu.async_copy(rowb[c_], env_out.at[pl.ds((gbase + j) * CH, CH), :],
                             semm[c_])

        @pl.when(c == 0)
        def _():
            step(0, 1)

        @pl.when(c == 1)
        def _():
            step(1, 0)

        return carry

    lax.fori_loop(0, ngch, gbody, 0)
    pltpu.make_async_copy(src2_hbm.at[gbase + ngch - 1], idx0, smi0).wait()
    pltpu.make_async_copy(rows0, env_out.at[pl.ds((gbase + ngch - 2) * CH, CH), :],
                          smm0).wait()
    pltpu.make_async_copy(rows1, env_out.at[pl.ds((gbase + ngch - 1) * CH, CH), :],
                          smm1).wait()


def _run_sc_env(msg, src2, zeros128):
    return pl.kernel(
        _sc_env,
        out_type=jax.ShapeDtypeStruct((EP, 128), jnp.float32),
        mesh=_MESH,
        scratch_types=[pltpu.VMEM((CH,), jnp.int32),
                       pltpu.VMEM((CH,), jnp.int32),
                       pltpu.VMEM((CH, 128), jnp.float32),
                       pltpu.VMEM((CH, 128), jnp.float32),
                       pltpu.VMEM((CH, 128), jnp.float32),
                       pltpu.VMEM_SHARED((NROWS, 128), jnp.float32),
                       pltpu.SemaphoreType.DMA,
                       pltpu.SemaphoreType.DMA,
                       pltpu.SemaphoreType.DMA,
                       pltpu.SemaphoreType.DMA,
                       pltpu.SemaphoreType.DMA],
    )(msg, src2, zeros128)


def _stage_a(fs_ref, fd_ref, w0_ref, wtype_ref, b0_ref, w1_ref, w2_ref,
             wsph_ref, wenv1_ref, xs_out, v_out, msg_out, aux_out):
    fs = fs_ref[...]
    fd = fd_ref[...]
    v0 = fd[:, 0:1] - fs[:, 0:1]
    v1 = fd[:, 1:2] - fs[:, 1:2]
    v2 = fd[:, 2:3] - fs[:, 2:3]
    r2 = v0 * v0 + v1 * v1 + v2 * v2
    r = jnp.sqrt(r2 + 1e-8)
    inv_r = 1.0 / r
    u = jnp.clip(r * (1.0 / RMAX), 0.0, 1.0)
    u2 = u * u
    u6 = u2 * u2 * u2
    u7 = u6 * u
    u8 = u7 * u
    fcut = 1.0 - 28.0 * u6 + 48.0 * u7 - 21.0 * u8
    n = (lax.broadcasted_iota(jnp.int32, (1, NB), 1) + 1).astype(jnp.float32)
    bess = jnp.sin(r * (n * (math.pi / RMAX))) * (math.sqrt(2.0 / RMAX) * inv_r)
    radial = bess * fcut
    oh_s = (fs[:, 3:4].astype(jnp.int32) == lax.broadcasted_iota(jnp.int32, (TE, NT), 1)).astype(jnp.float32)
    oh_d = (fd[:, 3:4].astype(jnp.int32) == lax.broadcasted_iota(jnp.int32, (TE, NT), 1)).astype(jnp.float32)
    emb_s = _dot_h(oh_s, wtype_ref[...])
    emb_d = _dot_h(oh_d, wtype_ref[...])
    feat0 = jnp.concatenate([emb_s, emb_d, radial], axis=1)
    h = _dot(feat0, w0_ref[...]) + b0_ref[...]
    x = _silu(h)
    x = _silu(_dot(x, w1_ref[...]))
    x = _silu(_dot(x, w2_ref[...]))
    wch = _dot(x, wsph_ref[...])
    e1 = _dot(x, wenv1_ref[...])
    v_full = jnp.concatenate([wch, wch * (v0 * inv_r), wch * (v1 * inv_r),
                              wch * (v2 * inv_r)], axis=1)
    msg = _tile4(e1) * v_full * (1.0 / 32.0)
    xs_out[...] = x
    v_out[...] = v_full
    msg_out[...] = msg
    aux_out[...] = fcut + jnp.zeros((TE, 8), jnp.float32)


def _stage_b(xs_ref, v_ref, env_ref, wlat_ref, wupd_ref, wenv2_ref,
             xs2_out, v2_out, msg2_out):
    x = xs_ref[...]
    v_full = v_ref[...]
    env = env_ref[...]
    prod = v_full * env
    inv = prod[:, 0:32] + prod[:, 32:64] + prod[:, 64:96] + prod[:, 96:128]
    x2 = x + _silu(_dot(jnp.concatenate([x, inv], axis=1), wlat_ref[...]))
    upd = _dot(x2, wupd_ref[...])
    v2 = v_full * _tile4(upd) + env
    ew2 = _dot(x2, wenv2_ref[...])
    msg2 = _tile4(ew2) * v2 * (1.0 / 32.0)
    xs2_out[...] = x2
    v2_out[...] = v2
    msg2_out[...] = msg2


def _stage_c(xs2_ref, v2_ref, env2_ref, fd_ref, aux_ref, wlat2_ref,
             wout_ref, scales_ref, acc_out):
    i = pl.program_id(0)
    x2 = xs2_ref[...]
    v2 = v2_ref[...]
    env2 = env2_ref[...]
    prod = v2 * env2
    inv2 = prod[:, 0:32] + prod[:, 32:64] + prod[:, 64:96] + prod[:, 96:128]
    x3 = x2 + _silu(_dot(jnp.concatenate([x2, inv2], axis=1), wlat2_ref[...]))
    e = _dot(x3, wout_ref[...])
    fcut = aux_ref[:, 0:1]
    td = fd_ref[:, 3:4]
    gd = fd_ref[:, 4:5]
    oh_t = (td.astype(jnp.int32) == lax.broadcasted_iota(jnp.int32, (TE, NT), 1)).astype(jnp.float32)
    scale_d = _dot_h(oh_t, scales_ref[...])
    ev = e * fcut * scale_d
    oh_g = (gd.astype(jnp.int32) == lax.broadcasted_iota(jnp.int32, (TE, NG), 1)).astype(jnp.float32)
    part = lax.dot_general(oh_g, ev, (((0,), (0,)), ((), ())),
                           preferred_element_type=jnp.float32,
                           precision=lax.Precision.HIGHEST)

    @pl.when(i == 0)
    def _():
        acc_out[...] = jnp.zeros((1, NG), jnp.float32)

    acc_out[...] += part.reshape(1, NG)


def _atom_shift(f_ref, shifts_ref, acc_out):
    i = pl.program_id(0)
    tf = f_ref[:, 3:4]
    gf = f_ref[:, 4:5]
    oh_t = (tf.astype(jnp.int32) == lax.broadcasted_iota(jnp.int32, (TN, NT), 1)).astype(jnp.float32)
    sval = _dot_h(oh_t, shifts_ref[...])
    oh_g = (gf.astype(jnp.int32) == lax.broadcasted_iota(jnp.int32, (TN, NG), 1)).astype(jnp.float32)
    part = lax.dot_general(oh_g, sval, (((0,), (0,)), ((), ())),
                           preferred_element_type=jnp.float32,
                           precision=lax.Precision.HIGHEST)

    @pl.when(i == 0)
    def _():
        acc_out[...] = jnp.zeros((1, NG), jnp.float32)

    acc_out[...] += part.reshape(1, NG)


def _edge_spec():
    return pl.BlockSpec((TE, 128), lambda i: (i, 0))


def _full(shape):
    return pl.BlockSpec(shape, lambda i: tuple(0 for _ in shape))


def _run_stage_a(fs, fd, w0, wtype, b0, w1, w2, wsph, wenv1):
    grid = (EP // TE,)
    return pl.pallas_call(
        _stage_a,
        grid=grid,
        in_specs=[pl.BlockSpec((TE, 16), lambda i: (i, 0)),
                  pl.BlockSpec((TE, 16), lambda i: (i, 0)),
                  _full((2 * 32 + NB, D)), _full((NT, 32)),
                  _full((1, D)), _full((D, D)), _full((D, D)),
                  _full((D, C)), _full((D, C))],
        out_specs=[_edge_spec(), _edge_spec(), _edge_spec(),
                   pl.BlockSpec((TE, 8), lambda i: (i, 0))],
        out_shape=[jax.ShapeDtypeStruct((EP, 128), jnp.float32),
                   jax.ShapeDtypeStruct((EP, 128), jnp.float32),
                   jax.ShapeDtypeStruct((EP, 128), jnp.float32),
                   jax.ShapeDtypeStruct((EP, 8), jnp.float32)],
    )(fs, fd, w0, wtype, b0, w1, w2, wsph, wenv1)


def _run_stage_b(xs, v, env3, wlat, wupd, wenv2):
    grid = (EP // TE,)
    return pl.pallas_call(
        _stage_b,
        grid=grid,
        in_specs=[_edge_spec(), _edge_spec(), _edge_spec(),
                  _full((D + C, D)), _full((D, C)), _full((D, C))],
        out_specs=[_edge_spec(), _edge_spec(), _edge_spec()],
        out_shape=[jax.ShapeDtypeStruct((EP, 128), jnp.float32),
                   jax.ShapeDtypeStruct((EP, 128), jnp.float32),
                   jax.ShapeDtypeStruct((EP, 128), jnp.float32)],
    )(xs, v, env3, wlat, wupd, wenv2)


def _run_stage_c(xs2, v2, env3, fd, aux, wlat2, wout, scales):
    grid = (EP // TE,)
    return pl.pallas_call(
        _stage_c,
        grid=grid,
        in_specs=[_edge_spec(), _edge_spec(), _edge_spec(),
                  pl.BlockSpec((TE, 16), lambda i: (i, 0)),
                  pl.BlockSpec((TE, 8), lambda i: (i, 0)),
                  _full((D + C, D)), _full((D, 1)), _full((NT, 1))],
        out_specs=pl.BlockSpec((1, NG), lambda i: (0, 0)),
        out_shape=jax.ShapeDtypeStruct((1, NG), jnp.float32),
    )(xs2, v2, env3, fd, aux, wlat2, wout, scales)


def _run_atom_shift(f_pad, shifts):
    grid = (NPAD // TN,)
    return pl.pallas_call(
        _atom_shift,
        grid=grid,
        in_specs=[pl.BlockSpec((TN, 16), lambda i: (i, 0)), _full((NT, 1))],
        out_specs=pl.BlockSpec((1, NG), lambda i: (0, 0)),
        out_shape=jax.ShapeDtypeStruct((1, NG), jnp.float32),
    )(f_pad, shifts)


def kernel(pos, W_type, W0, b0, W1, W2, W_sph, W_env1, W_lat1, W_upd1,
           W_env2, W_lat2, W_upd2, W_out, scales, shifts,
           edge_index, atom_types, batch):
    src = edge_index[0].astype(jnp.int32)
    dst = edge_index[1].astype(jnp.int32)

    tf = atom_types.astype(jnp.float32)[:, None]
    gf = batch.astype(jnp.float32)[:, None]
    f_tab = jnp.concatenate([pos, tf, gf, jnp.zeros((N, 11), jnp.float32)], axis=1)
    pad_rows = jnp.zeros((NPAD - N, 16), jnp.float32).at[:, 4].set(float(NG))
    f_pad = jnp.concatenate([f_tab, pad_rows], axis=0)

    padv = jnp.full((EP - E,), DUMMY, jnp.int32)
    src2 = jnp.concatenate([src, padv]).reshape(EP // CH, CH)
    dst2 = jnp.concatenate([dst, padv]).reshape(EP // CH, CH)
    zeros128 = jnp.zeros((CH, 128), jnp.float32)

    b0r = b0.reshape(1, D)

    fs, fd = _run_sc_gather(f_pad, src2, dst2)

    xs, v, msg1, aux = _run_stage_a(fs, fd, W0, W_type, b0r,
                                    W1, W2, W_sph, W_env1)

    env3_1 = _run_sc_env(msg1, src2, zeros128)

    xs2, v2, msg2 = _run_stage_b(xs, v, env3_1, W_lat1, W_upd1, W_env2)

    env3_2 = _run_sc_env(msg2, src2, zeros128)

    acc_e = _run_stage_c(xs2, v2, env3_2, fd, aux, W_lat2, W_out,
                         scales.reshape(NT, 1))
    acc_a = _run_atom_shift(f_pad, shifts.reshape(NT, 1))
    return (acc_e + acc_a).reshape(NG)


# speedup vs baseline: 1.1504x; 1.0212x over previous
import math

import jax
import jax.numpy as jnp
from jax import lax
from jax.experimental import pallas as pl
from jax.experimental.pallas import tpu as pltpu
from jax.experimental.pallas import tpu_sc as plsc

N = 10000
E = 320000
D = 128
C = 32
NT = 16
NB = 8
NG = 64
RMAX = 6.0

EP = 327680
CH = 128
EPW = EP // 32
NROWS = 10112
NPAD = 10240
DUMMY = N

TE = 1280
TN = 2048

_MESH = plsc.VectorSubcoreMesh(core_axis_name="c", subcore_axis_name="s")


def _silu(x):
    return x * (1.0 / (1.0 + jnp.exp(-x)))


def _dot(a, b):
    return jnp.dot(a, b, preferred_element_type=jnp.float32)


def _dot_h(a, b):
    return jnp.dot(a, b, preferred_element_type=jnp.float32,
                   precision=lax.Precision.HIGHEST)


def _tile4(x):
    return jnp.concatenate([x, x, x, x], axis=1)


def _sc_gather_feats(fpad_hbm, src2_hbm, dst2_hbm, fs_out, fd_out,
                     ixs0, ixs1, ixd0, ixd1, rs0, rs1, rd0, rd1,
                     sis0, sis1, sid0, sid1, sg1, sg2,
                     sos0, sos1, sod0, sod1):
    cid = lax.axis_index("c")
    sid = lax.axis_index("s")
    wid = cid * 16 + sid
    nchg = EPW // CH
    rbase = wid * nchg
    ixsb = (ixs0, ixs1)
    ixdb = (ixd0, ixd1)
    rsb = (rs0, rs1)
    rdb = (rd0, rd1)
    sisb = (sis0, sis1)
    sidb = (sid0, sid1)
    sosb = (sos0, sos1)
    sodb = (sod0, sod1)

    pltpu.async_copy(src2_hbm.at[rbase], ixs0, sis0)
    pltpu.async_copy(dst2_hbm.at[rbase], ixd0, sid0)

    def body(j, carry):
        c = lax.rem(j, 2)
        row_n = rbase + jnp.minimum(j + 1, nchg - 1)

        def step(c_, nx):
            pltpu.async_copy(src2_hbm.at[row_n], ixsb[nx], sisb[nx])
            pltpu.async_copy(dst2_hbm.at[row_n], ixdb[nx], sidb[nx])

            @pl.when(j >= 2)
            def _():
                eb2 = (rbase + j - 2) * CH
                pltpu.make_async_copy(rsb[c_], fs_out.at[pl.ds(eb2, CH), :],
                                      sosb[c_]).wait()
                pltpu.make_async_copy(rdb[c_], fd_out.at[pl.ds(eb2, CH), :],
                                      sodb[c_]).wait()

            pltpu.make_async_copy(src2_hbm.at[rbase + j], ixsb[c_], sisb[c_]).wait()
            pltpu.make_async_copy(dst2_hbm.at[rbase + j], ixdb[c_], sidb[c_]).wait()
            g1 = pltpu.async_copy(fpad_hbm.at[ixsb[c_]], rsb[c_], sg1)
            g2 = pltpu.async_copy(fpad_hbm.at[ixdb[c_]], rdb[c_], sg2)
            g1.wait()
            g2.wait()
            eb = (rbase + j) * CH
            pltpu.async_copy(rsb[c_], fs_out.at[pl.ds(eb, CH), :], sosb[c_])
            pltpu.async_copy(rdb[c_], fd_out.at[pl.ds(eb, CH), :], sodb[c_])

        @pl.when(c == 0)
        def _():
            step(0, 1)

        @pl.when(c == 1)
        def _():
            step(1, 0)

        return carry

    lax.fori_loop(0, nchg, body, 0)
    pltpu.make_async_copy(src2_hbm.at[rbase + nchg - 1], ixs0, sis0).wait()
    pltpu.make_async_copy(dst2_hbm.at[rbase + nchg - 1], ixd0, sid0).wait()
    pltpu.make_async_copy(rs0, fs_out.at[pl.ds((rbase + nchg - 2) * CH, CH), :],
                          sos0).wait()
    pltpu.make_async_copy(rd0, fd_out.at[pl.ds((rbase + nchg - 2) * CH, CH), :],
                          sod0).wait()
    pltpu.make_async_copy(rs1, fs_out.at[pl.ds((rbase + nchg - 1) * CH, CH), :],
                          sos1).wait()
    pltpu.make_async_copy(rd1, fd_out.at[pl.ds((rbase + nchg - 1) * CH, CH), :],
                          sod1).wait()


def _run_sc_gather(fpad, src2, dst2):
    return pl.kernel(
        _sc_gather_feats,
        out_type=[jax.ShapeDtypeStruct((EP, 16), jnp.float32),
                  jax.ShapeDtypeStruct((EP, 16), jnp.float32)],
        mesh=_MESH,
        compiler_params=pltpu.CompilerParams(use_tc_tiling_on_sc=False),
        scratch_types=[pltpu.VMEM((CH,), jnp.int32),
                       pltpu.VMEM((CH,), jnp.int32),
                       pltpu.VMEM((CH,), jnp.int32),
                       pltpu.VMEM((CH,), jnp.int32),
                       pltpu.VMEM((CH, 16), jnp.float32),
                       pltpu.VMEM((CH, 16), jnp.float32),
                       pltpu.VMEM((CH, 16), jnp.float32),
                       pltpu.VMEM((CH, 16), jnp.float32)]
                      + [pltpu.SemaphoreType.DMA] * 10,
    )(fpad, src2, dst2)


def _sc_env(msg_hbm, src2_hbm, zeros_hbm, env_out,
            idx0, idx1, rows0, rows1, zbuf, shared,
            smi0, smi1, smm0, smm1, sem):
    cid = lax.axis_index("c")
    sid = lax.axis_index("s")
    wid = cid * 16 + sid

    pltpu.sync_copy(zeros_hbm, zbuf)
    for k in range(-(-(NROWS // CH) // 16)):
        zc = sid + k * 16

        @pl.when(zc < NROWS // CH)
        def _():
            pltpu.sync_copy(zbuf, shared.at[pl.ds(zc * CH, CH), :])
    plsc.subcore_barrier()

    nch = (EP // CH) // 16
    rbase = sid * nch
    idxb = (idx0, idx1)
    rowb = (rows0, rows1)
    semi = (smi0, smi1)
    semm = (smm0, smm1)

    pltpu.async_copy(src2_hbm.at[rbase], idx0, smi0)
    pltpu.async_copy(msg_hbm.at[pl.ds(rbase * CH, CH), :], rows0, smm0)

    def sbody(j, carry):
        c = lax.rem(j, 2)
        row_n = rbase + jnp.minimum(j + 1, nch - 1)

        def pref(nx):
            pltpu.async_copy(src2_hbm.at[row_n], idxb[nx], semi[nx])
            pltpu.async_copy(msg_hbm.at[pl.ds(row_n * CH, CH), :], rowb[nx], semm[nx])

        def scat(c_):
            row_c = rbase + j
            pltpu.make_async_copy(src2_hbm.at[row_c], idxb[c_], semi[c_]).wait()
            pltpu.make_async_copy(msg_hbm.at[pl.ds(row_c * CH, CH), :], rowb[c_], semm[c_]).wait()
            pltpu.sync_copy(rowb[c_], shared.at[idxb[c_]], add=True)

        @pl.when(c == 0)
        def _():
            pref(1)
            scat(0)

        @pl.when(c == 1)
        def _():
            pref(0)
            scat(1)

        return carry

    lax.fori_loop(0, nch, sbody, 0)
    pltpu.make_async_copy(src2_hbm.at[rbase + nch - 1], idx0, smi0).wait()
    pltpu.make_async_copy(msg_hbm.at[pl.ds((rbase + nch - 1) * CH, CH), :], rows0, smm0).wait()
    plsc.subcore_barrier()

    ngch = EPW // CH
    gbase = wid * ngch

    pltpu.async_copy(src2_hbm.at[gbase], idx0, smi0)

    def gbody(j, carry):
        c = lax.rem(j, 2)
        row_n = gbase + jnp.minimum(j + 1, ngch - 1)

        def step(c_, nx):
            pltpu.async_copy(src2_hbm.at[row_n], idxb[nx], semi[nx])

            @pl.when(j >= 2)
            def _():
                pltpu.make_async_copy(
                    rowb[c_], env_out.at[pl.ds((gbase + j - 2) * CH, CH), :],
                    semm[c_]).wait()

            pltpu.make_async_copy(src2_hbm.at[gbase + j], idxb[c_], semi[c_]).wait()
            pltpu.async_copy(shared.at[idxb[c_]], rowb[c_], sem).wait()
            pltpu.async_copy(rowb[c_], env_out.at[pl.ds((gbase + j) * CH, CH), :],
                             semm[c_])

        @pl.when(c == 0)
        def _():
            step(0, 1)

        @pl.when(c == 1)
        def _():
            step(1, 0)

        return carry

    lax.fori_loop(0, ngch, gbody, 0)
    pltpu.make_async_copy(src2_hbm.at[gbase + ngch - 1], idx0, smi0).wait()
    pltpu.make_async_copy(rows0, env_out.at[pl.ds((gbase + ngch - 2) * CH, CH), :],
                          smm0).wait()
    pltpu.make_async_copy(rows1, env_out.at[pl.ds((gbase + ngch - 1) * CH, CH), :],
                          smm1).wait()


def _run_sc_env(msg, src2, zeros128):
    return pl.kernel(
        _sc_env,
        out_type=jax.ShapeDtypeStruct((EP, 128), jnp.float32),
        mesh=_MESH,
        scratch_types=[pltpu.VMEM((CH,), jnp.int32),
                       pltpu.VMEM((CH,), jnp.int32),
                       pltpu.VMEM((CH, 128), jnp.float32),
                       pltpu.VMEM((CH, 128), jnp.float32),
                       pltpu.VMEM((CH, 128), jnp.float32),
                       pltpu.VMEM_SHARED((NROWS, 128), jnp.float32),
                       pltpu.SemaphoreType.DMA,
                       pltpu.SemaphoreType.DMA,
                       pltpu.SemaphoreType.DMA,
                       pltpu.SemaphoreType.DMA,
                       pltpu.SemaphoreType.DMA],
    )(msg, src2, zeros128)


def _stage_a(fs_ref, fd_ref, w0_ref, wtype_ref, b0_ref, w1_ref, w2_ref,
             wsph_ref, wenv1_ref, xs_out, v_out, msg_out, aux_out):
    fs = fs_ref[...]
    fd = fd_ref[...]
    v0 = fd[:, 0:1] - fs[:, 0:1]
    v1 = fd[:, 1:2] - fs[:, 1:2]
    v2 = fd[:, 2:3] - fs[:, 2:3]
    r2 = v0 * v0 + v1 * v1 + v2 * v2
    r = jnp.sqrt(r2 + 1e-8)
    inv_r = 1.0 / r
    u = jnp.clip(r * (1.0 / RMAX), 0.0, 1.0)
    u2 = u * u
    u6 = u2 * u2 * u2
    u7 = u6 * u
    u8 = u7 * u
    fcut = 1.0 - 28.0 * u6 + 48.0 * u7 - 21.0 * u8
    n = (lax.broadcasted_iota(jnp.int32, (1, NB), 1) + 1).astype(jnp.float32)
    bess = jnp.sin(r * (n * (math.pi / RMAX))) * (math.sqrt(2.0 / RMAX) * inv_r)
    radial = bess * fcut
    oh_s = (fs[:, 3:4].astype(jnp.int32) == lax.broadcasted_iota(jnp.int32, (TE, NT), 1)).astype(jnp.float32)
    oh_d = (fd[:, 3:4].astype(jnp.int32) == lax.broadcasted_iota(jnp.int32, (TE, NT), 1)).astype(jnp.float32)
    emb_s = _dot_h(oh_s, wtype_ref[...])
    emb_d = _dot_h(oh_d, wtype_ref[...])
    feat0 = jnp.concatenate([emb_s, emb_d, radial], axis=1)
    h = _dot(feat0, w0_ref[...]) + b0_ref[...]
    x = _silu(h)
    x = _silu(_dot(x, w1_ref[...]))
    x = _silu(_dot(x, w2_ref[...]))
    wch = _dot(x, wsph_ref[...])
    e1 = _dot(x, wenv1_ref[...])
    v_full = jnp.concatenate([wch, wch * (v0 * inv_r), wch * (v1 * inv_r),
                              wch * (v2 * inv_r)], axis=1)
    msg = _tile4(e1) * v_full * (1.0 / 32.0)
    xs_out[...] = x
    v_out[...] = v_full
    msg_out[...] = msg
    aux_out[...] = fcut + jnp.zeros((TE, 8), jnp.float32)


def _stage_b(xs_ref, v_ref, env_ref, wlat_ref, wupd_ref, wenv2_ref,
             xs2_out, v2_out, msg2_out):
    x = xs_ref[...]
    v_full = v_ref[...]
    env = env_ref[...]
    prod = v_full * env
    inv = prod[:, 0:32] + prod[:, 32:64] + prod[:, 64:96] + prod[:, 96:128]
    x2 = x + _silu(_dot(jnp.concatenate([x, inv], axis=1), wlat_ref[...]))
    upd = _dot(x2, wupd_ref[...])
    v2 = v_full * _tile4(upd) + env
    ew2 = _dot(x2, wenv2_ref[...])
    msg2 = _tile4(ew2) * v2 * (1.0 / 32.0)
    xs2_out[...] = x2
    v2_out[...] = v2
    msg2_out[...] = msg2


def _stage_c(xs2_ref, v2_ref, env2_ref, fd_ref, aux_ref, wlat2_ref,
             wout_ref, scales_ref, acc_out):
    i = pl.program_id(0)
    x2 = xs2_ref[...]
    v2 = v2_ref[...]
    env2 = env2_ref[...]
    prod = v2 * env2
    inv2 = prod[:, 0:32] + prod[:, 32:64] + prod[:, 64:96] + prod[:, 96:128]
    x3 = x2 + _silu(_dot(jnp.concatenate([x2, inv2], axis=1), wlat2_ref[...]))
    e = _dot(x3, wout_ref[...])
    fcut = aux_ref[:, 0:1]
    td = fd_ref[:, 3:4]
    gd = fd_ref[:, 4:5]
    oh_t = (td.astype(jnp.int32) == lax.broadcasted_iota(jnp.int32, (TE, NT), 1)).astype(jnp.float32)
    scale_d = _dot_h(oh_t, scales_ref[...])
    ev = e * fcut * scale_d
    oh_g = (gd.astype(jnp.int32) == lax.broadcasted_iota(jnp.int32, (TE, NG), 1)).astype(jnp.float32)
    part = lax.dot_general(oh_g, ev, (((0,), (0,)), ((), ())),
                           preferred_element_type=jnp.float32,
                           precision=lax.Precision.HIGHEST)

    @pl.when(i == 0)
    def _():
        acc_out[...] = jnp.zeros((1, NG), jnp.float32)

    acc_out[...] += part.reshape(1, NG)


def _atom_shift(f_ref, shifts_ref, acc_out):
    i = pl.program_id(0)
    tf = f_ref[:, 3:4]
    gf = f_ref[:, 4:5]
    oh_t = (tf.astype(jnp.int32) == lax.broadcasted_iota(jnp.int32, (TN, NT), 1)).astype(jnp.float32)
    sval = _dot_h(oh_t, shifts_ref[...])
    oh_g = (gf.astype(jnp.int32) == lax.broadcasted_iota(jnp.int32, (TN, NG), 1)).astype(jnp.float32)
    part = lax.dot_general(oh_g, sval, (((0,), (0,)), ((), ())),
                           preferred_element_type=jnp.float32,
                           precision=lax.Precision.HIGHEST)

    @pl.when(i == 0)
    def _():
        acc_out[...] = jnp.zeros((1, NG), jnp.float32)

    acc_out[...] += part.reshape(1, NG)


def _edge_spec():
    return pl.BlockSpec((TE, 128), lambda i: (i, 0))


def _full(shape):
    return pl.BlockSpec(shape, lambda i: tuple(0 for _ in shape))


def _run_stage_a(fs, fd, w0, wtype, b0, w1, w2, wsph, wenv1):
    grid = (EP // TE,)
    return pl.pallas_call(
        _stage_a,
        grid=grid,
        in_specs=[pl.BlockSpec((TE, 16), lambda i: (i, 0)),
                  pl.BlockSpec((TE, 16), lambda i: (i, 0)),
                  _full((2 * 32 + NB, D)), _full((NT, 32)),
                  _full((1, D)), _full((D, D)), _full((D, D)),
                  _full((D, C)), _full((D, C))],
        out_specs=[_edge_spec(), _edge_spec(), _edge_spec(),
                   pl.BlockSpec((TE, 8), lambda i: (i, 0))],
        out_shape=[jax.ShapeDtypeStruct((EP, 128), jnp.float32),
                   jax.ShapeDtypeStruct((EP, 128), jnp.float32),
                   jax.ShapeDtypeStruct((EP, 128), jnp.float32),
                   jax.ShapeDtypeStruct((EP, 8), jnp.float32)],
    )(fs, fd, w0, wtype, b0, w1, w2, wsph, wenv1)


def _run_stage_b(xs, v, env3, wlat, wupd, wenv2):
    grid = (EP // TE,)
    return pl.pallas_call(
        _stage_b,
        grid=grid,
        in_specs=[_edge_spec(), _edge_spec(), _edge_spec(),
                  _full((D + C, D)), _full((D, C)), _full((D, C))],
        out_specs=[_edge_spec(), _edge_spec(), _edge_spec()],
        out_shape=[jax.ShapeDtypeStruct((EP, 128), jnp.float32),
                   jax.ShapeDtypeStruct((EP, 128), jnp.float32),
                   jax.ShapeDtypeStruct((EP, 128), jnp.float32)],
    )(xs, v, env3, wlat, wupd, wenv2)


def _run_stage_c(xs2, v2, env3, fd, aux, wlat2, wout, scales):
    grid = (EP // TE,)
    return pl.pallas_call(
        _stage_c,
        grid=grid,
        in_specs=[_edge_spec(), _edge_spec(), _edge_spec(),
                  pl.BlockSpec((TE, 16), lambda i: (i, 0)),
                  pl.BlockSpec((TE, 8), lambda i: (i, 0)),
                  _full((D + C, D)), _full((D, 1)), _full((NT, 1))],
        out_specs=pl.BlockSpec((1, NG), lambda i: (0, 0)),
        out_shape=jax.ShapeDtypeStruct((1, NG), jnp.float32),
    )(xs2, v2, env3, fd, aux, wlat2, wout, scales)


def _run_atom_shift(f_pad, shifts):
    grid = (NPAD // TN,)
    return pl.pallas_call(
        _atom_shift,
        grid=grid,
        in_specs=[pl.BlockSpec((TN, 16), lambda i: (i, 0)), _full((NT, 1))],
        out_specs=pl.BlockSpec((1, NG), lambda i: (0, 0)),
        out_shape=jax.ShapeDtypeStruct((1, NG), jnp.float32),
    )(f_pad, shifts)


def kernel(pos, W_type, W0, b0, W1, W2, W_sph, W_env1, W_lat1, W_upd1,
           W_env2, W_lat2, W_upd2, W_out, scales, shifts,
           edge_index, atom_types, batch):
    src = edge_index[0].astype(jnp.int32)
    dst = edge_index[1].astype(jnp.int32)

    tf = atom_types.astype(jnp.float32)[:, None]
    gf = batch.astype(jnp.float32)[:, None]
    f_tab = jnp.concatenate([pos, tf, gf, jnp.zeros((N, 11), jnp.float32)], axis=1)
    pad_rows = jnp.zeros((NPAD - N, 16), jnp.float32).at[:, 4].set(float(NG))
    f_pad = jnp.concatenate([f_tab, pad_rows], axis=0)

    padv = jnp.full((EP - E,), DUMMY, jnp.int32)
    src2 = jnp.concatenate([src, padv]).reshape(EP // CH, CH)
    dst2 = jnp.concatenate([dst, padv]).reshape(EP // CH, CH)
    zeros128 = jnp.zeros((CH, 128), jnp.float32)

    b0r = b0.reshape(1, D)

    fs, fd = _run_sc_gather(f_pad, src2, dst2)

    xs, v, msg1, aux = _run_stage_a(fs, fd, W0, W_type, b0r,
                                    W1, W2, W_sph, W_env1)

    env3_1 = _run_sc_env(msg1, src2, zeros128)

    xs2, v2, msg2 = _run_stage_b(xs, v, env3_1, W_lat1, W_upd1, W_env2)

    env3_2 = _run_sc_env(msg2, src2, zeros128)

    acc_e = _run_stage_c(xs2, v2, env3_2, fd, aux, W_lat2, W_out,
                         scales.reshape(NT, 1))
    acc_a = _run_atom_shift(f_pad, shifts.reshape(NT, 1))
    return (acc_e + acc_a).reshape(NG)
